# Initial kernel scaffold; baseline (speedup 1.0000x reference)
import jax
import jax.numpy as jnp
from jax.experimental import pallas as pl


def kernel(x_white, x_black, W_emb, bias1, fc2_W, fc2_b, cp_W, cp_b, wdl_W, wdl_b):
    raise NotImplementedError("write your pallas kernel here")


# speedup vs baseline: 1.8437x; 1.8437x over previous
import functools

import jax
import jax.numpy as jnp
from jax import lax
from jax.experimental import pallas as pl
from jax.experimental.pallas import tpu as pltpu
from jax.experimental.pallas import tpu_sc as plsc

B, L, V, H, H2 = 4096, 32, 12288, 512, 32
NC, NS, LANES = 2, 16, 16
NW = NC * NS
BT = 2 * B
BAGS_PER_W = BT // NW
GRP = 32
NGRP = BAGS_PER_W // GRP
CCHUNKS = H // LANES


_sc_mesh = plsc.VectorSubcoreMesh(
    core_axis_name="c", subcore_axis_name="s", num_cores=NC, num_subcores=NS
)


@functools.partial(
    pl.kernel,
    out_type=jax.ShapeDtypeStruct((BT, H), jnp.float32),
    mesh=_sc_mesh,
    scratch_types=[
        pltpu.VMEM((BAGS_PER_W, L), jnp.int32),
        pltpu.VMEM((L, H), jnp.float32),
        pltpu.VMEM((GRP, H), jnp.float32),
        pltpu.SemaphoreType.DMA,
    ],
)
def _pooling_sc(w_hbm, x_hbm, out_hbm, idx_v, rows_v, outg_v, sem):
    wid = lax.axis_index("s") * NC + lax.axis_index("c")
    base = wid * BAGS_PER_W
    pltpu.sync_copy(x_hbm.at[pl.ds(base, BAGS_PER_W)], idx_v)

    @pl.loop(0, NGRP)
    def _grp(grp):
        gbase = grp * GRP

        @pl.loop(0, GRP)
        def _bag(j):
            g = gbase + j
            pltpu.async_copy(w_hbm.at[idx_v.at[g]], rows_v, sem).wait()

            @pl.loop(0, CCHUNKS)
            def _col(c):
                col = c * LANES
                s = rows_v[0, pl.ds(col, LANES)]
                for r in range(1, L):
                    s = s + rows_v[r, pl.ds(col, LANES)]
                outg_v[j, pl.ds(col, LANES)] = s

        pltpu.sync_copy(outg_v, out_hbm.at[pl.ds(base + gbase, GRP)])


def _tail_tc(pw_ref, pb_ref, b1_ref, w2w_ref, w2b_ref, b2_ref, w4_ref, b4_ref,
             out_ref):
    hw = jnp.maximum(pw_ref[...] + b1_ref[...], 0.0)
    hb = jnp.maximum(pb_ref[...] + b1_ref[...], 0.0)
    h = jnp.dot(hw, w2w_ref[...], preferred_element_type=jnp.float32)
    h += jnp.dot(hb, w2b_ref[...], preferred_element_type=jnp.float32)
    h = jnp.maximum(h + b2_ref[...], 0.0)
    out_ref[...] = (
        jnp.dot(h, w4_ref[...], preferred_element_type=jnp.float32) + b4_ref[...]
    )


def kernel(x_white, x_black, W_emb, bias1, fc2_W, fc2_b, cp_W, cp_b, wdl_W, wdl_b):
    x_all = jnp.concatenate([x_white, x_black], axis=0).astype(jnp.int32)
    pooled = _pooling_sc(W_emb, x_all)

    w2w = fc2_W[:, :H].T
    w2b = fc2_W[:, H:].T
    w4 = jnp.concatenate([cp_W, wdl_W], axis=0).T
    b4 = jnp.concatenate([cp_b, wdl_b], axis=0)[None]

    bs = 512
    out4 = pl.pallas_call(
        _tail_tc,
        grid=(B // bs,),
        in_specs=[
            pl.BlockSpec((bs, H), lambda i: (i, 0)),
            pl.BlockSpec((bs, H), lambda i: (i + B // bs, 0)),
            pl.BlockSpec((1, H), lambda i: (0, 0)),
            pl.BlockSpec((H, H2), lambda i: (0, 0)),
            pl.BlockSpec((H, H2), lambda i: (0, 0)),
            pl.BlockSpec((1, H2), lambda i: (0, 0)),
            pl.BlockSpec((H2, 4), lambda i: (0, 0)),
            pl.BlockSpec((1, 4), lambda i: (0, 0)),
        ],
        out_specs=pl.BlockSpec((bs, 4), lambda i: (i, 0)),
        out_shape=jax.ShapeDtypeStruct((B, 4), jnp.float32),
    )(pooled, pooled, bias1[None], w2w, w2b, fc2_b[None], w4, b4)

    return out4[:, 0:1], out4[:, 1:4]


# speedup vs baseline: 3.8642x; 2.0959x over previous
import functools

import jax
import jax.numpy as jnp
from jax import lax
from jax.experimental import pallas as pl
from jax.experimental.pallas import tpu as pltpu
from jax.experimental.pallas import tpu_sc as plsc

B, L, V, H, H2 = 4096, 32, 12288, 512, 32
NC, NS, LANES = 2, 16, 16
NW = NC * NS
BT = 2 * B
BAGS_PER_W = BT // NW
GRP = 32
NGRP = BAGS_PER_W // GRP
CCHUNKS = H // LANES


_sc_mesh = plsc.VectorSubcoreMesh(
    core_axis_name="c", subcore_axis_name="s", num_cores=NC, num_subcores=NS
)


@functools.partial(
    pl.kernel,
    out_type=jax.ShapeDtypeStruct((BT, H), jnp.float32),
    mesh=_sc_mesh,
    scratch_types=[
        pltpu.VMEM((BAGS_PER_W, L), jnp.int32),
        pltpu.VMEM((2, L, H), jnp.float32),
        pltpu.VMEM((GRP, H), jnp.float32),
        pltpu.SemaphoreType.DMA,
        pltpu.SemaphoreType.DMA,
    ],
)
def _pooling_sc(w_hbm, x_hbm, out_hbm, idx_v, rows_v, outg_v, sem0, sem1):
    wid = lax.axis_index("s") * NC + lax.axis_index("c")
    base = wid * BAGS_PER_W
    sems = (sem0, sem1)
    pltpu.sync_copy(x_hbm.at[pl.ds(base, BAGS_PER_W)], idx_v)

    def _start(g, b):
        pltpu.async_copy(w_hbm.at[idx_v.at[g]], rows_v.at[b], sems[b])

    def _wait(b):
        pltpu.make_async_copy(w_hbm.at[idx_v.at[0]], rows_v.at[b], sems[b]).wait()

    def _acc(b, j):
        @pl.loop(0, CCHUNKS)
        def _col(c):
            col = c * LANES
            vals = [rows_v[b, r, pl.ds(col, LANES)] for r in range(L)]
            while len(vals) > 1:
                nxt = [vals[i] + vals[i + 1] for i in range(0, len(vals) - 1, 2)]
                if len(vals) % 2:
                    nxt.append(vals[-1])
                vals = nxt
            outg_v[j, pl.ds(col, LANES)] = vals[0]

    _start(0, 0)

    @pl.loop(0, NGRP)
    def _grp(grp):
        gbase = grp * GRP

        @pl.loop(0, GRP, step=2)
        def _pair(j):
            g = gbase + j
            _start(g + 1, 1)
            _wait(0)
            _acc(0, j)

            @pl.when(g + 2 < BAGS_PER_W)
            def _():
                _start(g + 2, 0)

            _wait(1)
            _acc(1, j + 1)

        pltpu.sync_copy(outg_v, out_hbm.at[pl.ds(base + gbase, GRP)])


def _tail_tc(pw_ref, pb_ref, b1_ref, w2w_ref, w2b_ref, b2_ref, w4_ref, b4_ref,
             out_ref):
    hw = jnp.maximum(pw_ref[...] + b1_ref[...], 0.0)
    hb = jnp.maximum(pb_ref[...] + b1_ref[...], 0.0)
    h = jnp.dot(hw, w2w_ref[...], preferred_element_type=jnp.float32)
    h += jnp.dot(hb, w2b_ref[...], preferred_element_type=jnp.float32)
    h = jnp.maximum(h + b2_ref[...], 0.0)
    out_ref[...] = (
        jnp.dot(h, w4_ref[...], preferred_element_type=jnp.float32) + b4_ref[...]
    )


def kernel(x_white, x_black, W_emb, bias1, fc2_W, fc2_b, cp_W, cp_b, wdl_W, wdl_b):
    x_all = jnp.concatenate([x_white, x_black], axis=0).astype(jnp.int32)
    pooled = _pooling_sc(W_emb, x_all)

    w2w = fc2_W[:, :H].T
    w2b = fc2_W[:, H:].T
    w4 = jnp.concatenate([cp_W, wdl_W], axis=0).T
    b4 = jnp.concatenate([cp_b, wdl_b], axis=0)[None]

    bs = 512
    out4 = pl.pallas_call(
        _tail_tc,
        grid=(B // bs,),
        in_specs=[
            pl.BlockSpec((bs, H), lambda i: (i, 0)),
            pl.BlockSpec((bs, H), lambda i: (i + B // bs, 0)),
            pl.BlockSpec((1, H), lambda i: (0, 0)),
            pl.BlockSpec((H, H2), lambda i: (0, 0)),
            pl.BlockSpec((H, H2), lambda i: (0, 0)),
            pl.BlockSpec((1, H2), lambda i: (0, 0)),
            pl.BlockSpec((H2, 4), lambda i: (0, 0)),
            pl.BlockSpec((1, 4), lambda i: (0, 0)),
        ],
        out_specs=pl.BlockSpec((bs, 4), lambda i: (i, 0)),
        out_shape=jax.ShapeDtypeStruct((B, 4), jnp.float32),
    )(pooled, pooled, bias1[None], w2w, w2b, fc2_b[None], w4, b4)

    return out4[:, 0:1], out4[:, 1:4]


# speedup vs baseline: 4.4178x; 1.1433x over previous
import functools

import jax
import jax.numpy as jnp
from jax import lax
from jax.experimental import pallas as pl
from jax.experimental.pallas import tpu as pltpu
from jax.experimental.pallas import tpu_sc as plsc

B, L, V, H, H2 = 4096, 32, 12288, 512, 32
NC, NS, LANES = 2, 16, 16
NW = NC * NS
BT = 2 * B
BAGS_PER_W = BT // NW
GRP = 32
NGRP = BAGS_PER_W // GRP
CCHUNKS = H // LANES


_sc_mesh = plsc.VectorSubcoreMesh(
    core_axis_name="c", subcore_axis_name="s", num_cores=NC, num_subcores=NS
)


@functools.partial(
    pl.kernel,
    out_type=jax.ShapeDtypeStruct((BT, H), jnp.float32),
    mesh=_sc_mesh,
    scratch_types=[
        pltpu.VMEM((BAGS_PER_W * L,), jnp.int32),
        pltpu.VMEM((2, 2 * L, H), jnp.float32),
        pltpu.VMEM((2, GRP, H), jnp.float32),
        pltpu.SemaphoreType.DMA,
        pltpu.SemaphoreType.DMA,
        pltpu.SemaphoreType.DMA,
        pltpu.SemaphoreType.DMA,
    ],
)
def _pooling_sc(w_hbm, x_hbm, out_hbm, idx_v, rows_v, outg_v, sem0, sem1,
                osem0, osem1):
    wid = lax.axis_index("s") * NC + lax.axis_index("c")
    base = wid * BAGS_PER_W
    sems = (sem0, sem1)
    osems = (osem0, osem1)
    PAIRS = BAGS_PER_W // 2
    PGRP = GRP // 2
    pltpu.sync_copy(x_hbm.at[pl.ds(base * L, BAGS_PER_W * L)], idx_v)

    def _start(p, b):
        pltpu.async_copy(
            w_hbm.at[idx_v.at[pl.ds(p * 2 * L, 2 * L)]], rows_v.at[b], sems[b]
        )

    def _wait(b):
        pltpu.make_async_copy(
            w_hbm.at[idx_v.at[pl.ds(0, 2 * L)]], rows_v.at[b], sems[b]
        ).wait()

    def _acc2(b, ob, j):
        for half in range(2):
            @pl.loop(0, CCHUNKS)
            def _col(c):
                col = c * LANES
                vals = [
                    rows_v[b, half * L + r, pl.ds(col, LANES)] for r in range(L)
                ]
                while len(vals) > 1:
                    nxt = [vals[i] + vals[i + 1]
                           for i in range(0, len(vals) - 1, 2)]
                    if len(vals) % 2:
                        nxt.append(vals[-1])
                    vals = nxt
                outg_v[ob, j + half, pl.ds(col, LANES)] = vals[0]

    _start(0, 0)

    for g in range(NGRP):
        ob = g % 2
        if g >= 2:
            pltpu.make_async_copy(
                outg_v.at[ob], out_hbm.at[pl.ds(base, GRP)], osems[ob]
            ).wait()

        gp = g * PGRP

        @pl.loop(0, PGRP, step=2)
        def _pair(q):
            p = gp + q
            _start(p + 1, 1)
            _wait(0)
            _acc2(0, ob, 2 * q)

            @pl.when(p + 2 < PAIRS)
            def _():
                _start(p + 2, 0)

            _wait(1)
            _acc2(1, ob, 2 * q + 2)

        pltpu.async_copy(
            outg_v.at[ob], out_hbm.at[pl.ds(base + g * GRP, GRP)], osems[ob]
        )

    for ob in range(2):
        pltpu.make_async_copy(
            outg_v.at[ob], out_hbm.at[pl.ds(base, GRP)], osems[ob]
        ).wait()


def _tail_tc(pw_ref, pb_ref, b1_ref, w2w_ref, w2b_ref, b2_ref, w4_ref, b4_ref,
             out_ref):
    hw = jnp.maximum(pw_ref[...] + b1_ref[...], 0.0)
    hb = jnp.maximum(pb_ref[...] + b1_ref[...], 0.0)
    h = jnp.dot(hw, w2w_ref[...], preferred_element_type=jnp.float32)
    h += jnp.dot(hb, w2b_ref[...], preferred_element_type=jnp.float32)
    h = jnp.maximum(h + b2_ref[...], 0.0)
    out_ref[...] = (
        jnp.dot(h, w4_ref[...], preferred_element_type=jnp.float32) + b4_ref[...]
    )


def kernel(x_white, x_black, W_emb, bias1, fc2_W, fc2_b, cp_W, cp_b, wdl_W, wdl_b):
    x_all = jnp.concatenate([x_white, x_black], axis=0).astype(jnp.int32)
    pooled = _pooling_sc(W_emb, x_all.reshape(-1))

    w2w = fc2_W[:, :H].T
    w2b = fc2_W[:, H:].T
    w4 = jnp.concatenate([cp_W, wdl_W], axis=0).T
    b4 = jnp.concatenate([cp_b, wdl_b], axis=0)[None]

    bs = 512
    out4 = pl.pallas_call(
        _tail_tc,
        grid=(B // bs,),
        in_specs=[
            pl.BlockSpec((bs, H), lambda i: (i, 0)),
            pl.BlockSpec((bs, H), lambda i: (i + B // bs, 0)),
            pl.BlockSpec((1, H), lambda i: (0, 0)),
            pl.BlockSpec((H, H2), lambda i: (0, 0)),
            pl.BlockSpec((H, H2), lambda i: (0, 0)),
            pl.BlockSpec((1, H2), lambda i: (0, 0)),
            pl.BlockSpec((H2, 4), lambda i: (0, 0)),
            pl.BlockSpec((1, 4), lambda i: (0, 0)),
        ],
        out_specs=pl.BlockSpec((bs, 4), lambda i: (i, 0)),
        out_shape=jax.ShapeDtypeStruct((B, 4), jnp.float32),
    )(pooled, pooled, bias1[None], w2w, w2b, fc2_b[None], w4, b4)

    return out4[:, 0:1], out4[:, 1:4]
